# baseline (speedup 1.0000x reference)
import jax
import jax.numpy as jnp
from jax.experimental import pallas as pl
from jax.experimental.pallas import tpu as pltpu


def _stage1(x_ref, w_ref, s_ref):
    xt = x_ref[:, :].astype(jnp.bfloat16).T
    w = w_ref[:, :].astype(jnp.bfloat16)
    s = jnp.dot(xt, w, preferred_element_type=jnp.float32)
    s_ref[:, :] = s.astype(jnp.bfloat16)


def _stage2(adj_ref, s_ref, b_ref, out_ref):
    a = adj_ref[:, :].astype(jnp.bfloat16)
    acc = jnp.dot(a, s_ref[:, :], preferred_element_type=jnp.float32)
    acc = acc + b_ref[:, :]
    out_ref[:, :] = acc.T


def kernel(input, adj, weight, bias):
    C, N = input.shape
    F = weight.shape[1]

    s = pl.pallas_call(
        _stage1,
        in_specs=[
            pl.BlockSpec((C, N), lambda: (0, 0)),
            pl.BlockSpec((C, F), lambda: (0, 0)),
        ],
        out_specs=pl.BlockSpec((N, F), lambda: (0, 0)),
        out_shape=jax.ShapeDtypeStruct((N, F), jnp.bfloat16),
    )(input, weight)

    TN = 512
    bias2 = bias.reshape(1, F)
    out = pl.pallas_call(
        _stage2,
        grid=(pl.cdiv(N, TN),),
        in_specs=[
            pl.BlockSpec((TN, N), lambda i: (i, 0)),
            pl.BlockSpec((N, F), lambda i: (0, 0)),
            pl.BlockSpec((1, F), lambda i: (0, 0)),
        ],
        out_specs=pl.BlockSpec((F, TN), lambda i: (0, i)),
        out_shape=jax.ShapeDtypeStruct((F, N), jnp.float32),
    )(adj, s, bias2)
    return out


# speedup vs baseline: 1.0490x; 1.0490x over previous
import jax
import jax.numpy as jnp
from jax.experimental import pallas as pl
from jax.experimental.pallas import tpu as pltpu


def _make_fused(N, TN, SUB, G):
    NSUB = TN // SUB
    REM = N - (G - 1) * TN

    def _fused(x_ref, w_ref, b_ref, adj_hbm, out_ref, s_ref, abuf, sem):
        i = pl.program_id(0)

        def full_copies(block, slot):
            return [
                pltpu.make_async_copy(
                    adj_hbm.at[pl.ds(block * TN + k * SUB, SUB), :],
                    abuf.at[slot, pl.ds(k * SUB, SUB), :],
                    sem.at[slot],
                )
                for k in range(NSUB)
            ]

        def tail_copy(slot):
            return pltpu.make_async_copy(
                adj_hbm.at[pl.ds((G - 1) * TN, REM), :],
                abuf.at[slot, pl.ds(0, REM), :],
                sem.at[slot],
            )

        def issue(block, slot):
            @pl.when(block < G - 1)
            def _():
                for c in full_copies(block, slot):
                    c.start()

            @pl.when(block == G - 1)
            def _():
                tail_copy(slot).start()

        def wait(block, slot):
            @pl.when(block < G - 1)
            def _():
                for c in full_copies(block, slot):
                    c.wait()

            @pl.when(block == G - 1)
            def _():
                tail_copy(slot).wait()

        @pl.when(i == 0)
        def _():
            issue(0, 0)
            xt = x_ref[:, :].astype(jnp.bfloat16).T
            w = w_ref[:, :].astype(jnp.bfloat16)
            s = jnp.dot(xt, w, preferred_element_type=jnp.float32)
            s_ref[:, :] = s.astype(jnp.bfloat16)

        @pl.when(i + 1 < G)
        def _():
            issue(i + 1, (i + 1) % 2)

        wait(i, i % 2)

        slot = i % 2
        a = abuf[slot].astype(jnp.bfloat16)
        acc = jnp.dot(a, s_ref[:, :], preferred_element_type=jnp.float32)
        acc = acc + b_ref[:, :]
        out_ref[:, :] = acc.T

    return _fused


def kernel(input, adj, weight, bias):
    C, N = input.shape
    F = weight.shape[1]

    TN = 256
    SUB = 32
    G = pl.cdiv(N, TN)
    bias2 = bias.reshape(1, F)

    out = pl.pallas_call(
        _make_fused(N, TN, SUB, G),
        grid=(G,),
        in_specs=[
            pl.BlockSpec((C, N), lambda i: (0, 0)),
            pl.BlockSpec((C, F), lambda i: (0, 0)),
            pl.BlockSpec((1, F), lambda i: (0, 0)),
            pl.BlockSpec(memory_space=pl.ANY),
        ],
        out_specs=pl.BlockSpec((F, TN), lambda i: (0, i)),
        out_shape=jax.ShapeDtypeStruct((F, N), jnp.float32),
        scratch_shapes=[
            pltpu.VMEM((N, F), jnp.bfloat16),
            pltpu.VMEM((2, TN, N), jnp.float32),
            pltpu.SemaphoreType.DMA((2,)),
        ],
    )(input, weight, bias2, adj)
    return out
